# baseline (speedup 1.0000x reference)
import functools

import jax
import jax.numpy as jnp
from jax import lax
from jax.experimental import pallas as pl
from jax.experimental.pallas import tpu as pltpu
from jax.experimental.pallas import tpu_sc as plsc

TOP_K = 8
N_EXPERTS = 64
HIDDEN = 4096

BT = 1024
N_TOKENS = 16384
NW = 32
TPW = N_TOKENS // NW


def _logits_kernel(x_ref, w_in_ref, st_ref):
    logits = lax.dot_general(
        x_ref[...],
        w_in_ref[...],
        (((1,), (1,)), ((), ())),
        preferred_element_type=jnp.float32,
    )
    st_ref[...] = jax.nn.sigmoid(logits).T


def _scores_t(flat, w):
    return pl.pallas_call(
        _logits_kernel,
        grid=(N_TOKENS // BT,),
        in_specs=[
            pl.BlockSpec((BT, HIDDEN), lambda i: (i, 0)),
            pl.BlockSpec((N_EXPERTS, HIDDEN), lambda i: (0, 0)),
        ],
        out_specs=pl.BlockSpec((N_EXPERTS, BT), lambda i: (0, i)),
        out_shape=jax.ShapeDtypeStruct((N_EXPERTS, N_TOKENS), jnp.float32),
    )(flat, w)


_sc_mesh = plsc.VectorSubcoreMesh(core_axis_name="c", subcore_axis_name="s")


@functools.partial(
    pl.kernel,
    mesh=_sc_mesh,
    out_type=[
        jax.ShapeDtypeStruct((TOP_K, N_TOKENS), jnp.int32),
        jax.ShapeDtypeStruct((TOP_K, N_TOKENS), jnp.float32),
    ],
    scratch_types=[
        pltpu.VMEM((N_EXPERTS, TPW), jnp.float32),
        pltpu.VMEM((TOP_K, TPW), jnp.int32),
        pltpu.VMEM((TOP_K, TPW), jnp.float32),
    ],
)
def _topk_sc(st_hbm, idx_hbm, w_hbm, s_v, i_v, w_v):
    wid = lax.axis_index("s") * 2 + lax.axis_index("c")
    base = wid * TPW
    pltpu.sync_copy(st_hbm.at[:, pl.ds(base, TPW)], s_v)

    lane = lax.iota(jnp.int32, 16)

    def group(g, carry):
        col = g * 16
        vals = [jnp.full((16,), -1.0, jnp.float32) for _ in range(TOP_K)]
        ids = [jnp.zeros((16,), jnp.int32) for _ in range(TOP_K)]
        for e in range(N_EXPERTS):
            cur_v = s_v[e, pl.ds(col, 16)]
            cur_i = jnp.full((16,), e, jnp.int32)
            for j in range(TOP_K):
                take = cur_v > vals[j]
                nv = jnp.where(take, cur_v, vals[j])
                ni = jnp.where(take, cur_i, ids[j])
                cv = jnp.where(take, vals[j], cur_v)
                ci = jnp.where(take, ids[j], cur_i)
                vals[j], ids[j] = nv, ni
                cur_v, cur_i = cv, ci
        tot = vals[0]
        for j in range(1, TOP_K):
            tot = tot + vals[j]
        tot = tot + 1e-20
        for k in range(TOP_K):
            i_v[k, pl.ds(col, 16)] = ids[k]
            w_v[k, pl.ds(col, 16)] = vals[k] / tot
        return carry

    lax.fori_loop(0, TPW // 16, group, 0)
    pltpu.sync_copy(i_v, idx_hbm.at[:, pl.ds(base, TPW)])
    pltpu.sync_copy(w_v, w_hbm.at[:, pl.ds(base, TPW)])


@jax.jit
def _gate(flat, w):
    st = _scores_t(flat, w)
    idx_t, w_t = _topk_sc(st)
    return (idx_t.T, w_t.T)


def kernel(hidden_states, W):
    bsz, seq_len, h = hidden_states.shape
    flat = hidden_states.reshape(-1, h)
    topk_idx, topk_weight = _gate(flat, W)
    return (topk_idx, topk_weight)


# speedup vs baseline: 1.2002x; 1.2002x over previous
import jax
import jax.numpy as jnp
from jax.experimental import pallas as pl

TOP_K = 8
N_EXPERTS = 64
HIDDEN = 4096

BT = 1024


def _gate_kernel(x0_ref, x1_ref, w_ref_in, idx_ref, w_ref):
    w_all = w_ref_in[...]
    logits = jax.lax.dot_general(
        x0_ref[...],
        w_all[:, : HIDDEN // 2],
        (((1,), (1,)), ((), ())),
        preferred_element_type=jnp.float32,
    ) + jax.lax.dot_general(
        x1_ref[...],
        w_all[:, HIDDEN // 2 :],
        (((1,), (1,)), ((), ())),
        preferred_element_type=jnp.float32,
    )
    s = jax.nn.sigmoid(logits).T

    iota = jax.lax.broadcasted_iota(jnp.int32, (N_EXPERTS, BT), 0).astype(
        jnp.float32
    )
    vals = []
    idxs = []
    for _ in range(TOP_K):
        m = jnp.max(s, axis=0, keepdims=True)
        hit = s >= m
        idx = jnp.min(jnp.where(hit, iota, float(N_EXPERTS)), axis=0, keepdims=True)
        vals.append(m)
        idxs.append(idx)
        s = jnp.where(iota == idx, -1.0, s)

    topv = jnp.concatenate(vals, axis=0)
    topi = jnp.concatenate(idxs, axis=0)
    denom = jnp.sum(topv, axis=0, keepdims=True) + 1e-20
    idx_ref[...] = topi.T.astype(jnp.int32)
    w_ref[...] = (topv / denom).T


@jax.jit
def _gate(flat, w):
    n_tokens = flat.shape[0]
    grid = (n_tokens // BT,)
    return pl.pallas_call(
        _gate_kernel,
        grid=grid,
        in_specs=[
            pl.BlockSpec((BT, HIDDEN // 2), lambda i: (i, 0)),
            pl.BlockSpec((BT, HIDDEN // 2), lambda i: (i, 1)),
            pl.BlockSpec((N_EXPERTS, HIDDEN), lambda i: (0, 0)),
        ],
        out_specs=[
            pl.BlockSpec((BT, TOP_K), lambda i: (i, 0)),
            pl.BlockSpec((BT, TOP_K), lambda i: (i, 0)),
        ],
        out_shape=[
            jax.ShapeDtypeStruct((n_tokens, TOP_K), jnp.int32),
            jax.ShapeDtypeStruct((n_tokens, TOP_K), jnp.float32),
        ],
    )(flat, flat, w)


def kernel(hidden_states, W):
    bsz, seq_len, h = hidden_states.shape
    flat = hidden_states.reshape(-1, h)
    topk_idx, topk_weight = _gate(flat, W)
    return (topk_idx, topk_weight)
